# baseline (speedup 1.0000x reference)
import jax
import jax.numpy as jnp
from jax import lax
from jax.experimental import pallas as pl
from jax.experimental.pallas import tpu as pltpu
from jax.experimental.pallas import tpu_sc as plsc

N_COL = 10000
N_TAB = 500
N_NUM = 30000
SEQ = 16
H = 768
C = 16
E_TC = 10000
E_CC = 64000
E_NC = 30000

NC, NS, L = 2, 16, 16
NW = NC * NS

N_SEQ = N_COL + N_TAB
N_SEQ_PAD = 10752
OUT_SUB = 16
NBUF = 4
SEQ_SC0 = 352
SEQ_SC1 = 320

OFF_TAB = 0
OFF_COL = 512
OFF_NUM = 10512
OFF_ITC = 40512
OFF_ICC = 50512
OFF_INC = 60512
DEG_TOT = 70656
DEG_SLICE = DEG_TOT // NS
DEG_TRASH = 70600

BATCH = 128
E_DEG = 2 * (E_TC + E_CC + E_NC)
DEG_K = 51

H_ROWS = 40512
ACC_RELOFF = 10112
ACC_ROWS = 3 * ACC_RELOFF
ACC_TRASH = 10008
ACC_SLICE = ACC_ROWS // NS
E_MSG = E_TC + E_CC + E_NC
MSG_K = 26

f32 = jnp.float32
i32 = jnp.int32


def _sc_pool_deg_body(ids_hbm, emb_hbm, degidx_hbm,
                      out_hbm, degp_hbm,
                      idx_v, tok0, tok1, tok2, tok3, obuf0, obuf1,
                      didx_v, ones_v, degbuf,
                      sem0, sem1, sem2, sem3, osem0, osem1, dsem, deg_sh):
    cid = lax.axis_index("c")
    sid = lax.axis_index("s")
    wid = sid * NC + cid

    zero16 = jnp.zeros((L,), f32)

    def zfill(r, carry):
        degbuf[pl.ds(r * L, L)] = zero16
        return carry

    lax.fori_loop(0, DEG_SLICE // L, zfill, 0)
    pltpu.sync_copy(degbuf, deg_sh.at[pl.ds(sid * DEG_SLICE, DEG_SLICE)])
    plsc.subcore_barrier()

    ones16 = jnp.full((L,), 1.0, f32)
    for i in range(BATCH // L):
        ones_v[pl.ds(i * L, L)] = ones16

    pltpu.sync_copy(degidx_hbm.at[wid], didx_v)
    for j in range(DEG_K):
        pltpu.async_copy(ones_v, deg_sh.at[didx_v.at[j]], dsem, add=True)
    for j in range(DEG_K):
        pltpu.make_async_copy(ones_v, deg_sh.at[didx_v.at[j]], dsem).wait()
    plsc.subcore_barrier()
    pltpu.sync_copy(deg_sh.at[pl.ds(sid * DEG_SLICE, DEG_SLICE)], degbuf)
    pltpu.sync_copy(degbuf,
                    degp_hbm.at[pl.ds(cid * DEG_TOT + sid * DEG_SLICE,
                                      DEG_SLICE)])

    toks = [tok0, tok1, tok2, tok3]
    sems = [sem0, sem1, sem2, sem3]
    obufs = [obuf0, obuf1]
    osems = [osem0, osem1]
    himask = jnp.full((L,), -65536, i32)
    sh16 = jnp.full((L,), 16, i32)

    def pool_part(base, nseq):
        n_pairs = nseq // (2 * OUT_SUB)
        pltpu.sync_copy(ids_hbm.at[pl.ds(base, nseq)],
                        idx_v.at[pl.ds(0, nseq)])
        for b in range(NBUF):
            pltpu.async_copy(emb_hbm.at[idx_v.at[b]], toks[b], sems[b])

        def pair_body(p, carry):
            for e in range(2):
                outbuf = obufs[e]
                osem = osems[e]

                @pl.when(p >= 1)
                def _():
                    pltpu.make_async_copy(
                        outbuf, out_hbm.at[pl.ds(base, OUT_SUB)],
                        osem).wait()

                def quad_body(g, carry2, e=e, outbuf=outbuf):
                    for b in range(NBUF):
                        s = (p * 2 + e) * OUT_SUB + g * NBUF + b
                        tok = toks[b]
                        sem = sems[b]
                        pltpu.make_async_copy(emb_hbm.at[idx_v.at[s]], tok,
                                              sem).wait()
                        srow = g * NBUF + b

                        def col_body(k, carry3):
                            acc_lo = jnp.zeros((L,), f32)
                            acc_hi = jnp.zeros((L,), f32)
                            for j in range(SEQ):
                                vi = tok[j, pl.ds(k * L, L)]
                                acc_lo = acc_lo + plsc.bitcast(
                                    lax.shift_left(vi, sh16), f32)
                                acc_hi = acc_hi + plsc.bitcast(
                                    vi & himask, f32)
                            outbuf[srow, pl.ds(k * L, L)] = acc_lo
                            outbuf[srow, pl.ds(H // 2 + k * L, L)] = acc_hi
                            return carry3

                        lax.fori_loop(0, H // (2 * L), col_body, 0)

                        @pl.when(s + NBUF < nseq)
                        def _():
                            pltpu.async_copy(
                                emb_hbm.at[idx_v.at[s + NBUF]], tok, sem)
                    return carry2

                lax.fori_loop(0, OUT_SUB // NBUF, quad_body, 0)
                pltpu.async_copy(
                    outbuf,
                    out_hbm.at[pl.ds(base + (p * 2 + e) * OUT_SUB,
                                     OUT_SUB)],
                    osem)
            return carry

        lax.fori_loop(0, n_pairs, pair_body, 0)
        for e in range(2):
            pltpu.make_async_copy(obufs[e],
                                  out_hbm.at[pl.ds(base, OUT_SUB)],
                                  osems[e]).wait()

    @pl.when(cid == 0)
    def _():
        pool_part(sid * SEQ_SC0, SEQ_SC0)

    @pl.when(cid == 1)
    def _():
        pool_part(NS * SEQ_SC0 + sid * SEQ_SC1, SEQ_SC1)


def _make_sc_pool_deg():
    mesh = plsc.VectorSubcoreMesh(core_axis_name="c", subcore_axis_name="s")
    return pl.kernel(
        _sc_pool_deg_body,
        out_type=[jax.ShapeDtypeStruct((N_SEQ_PAD, H), f32),
                  jax.ShapeDtypeStruct((NC * DEG_TOT,), f32)],
        mesh=mesh,
        compiler_params=pltpu.CompilerParams(needs_layout_passes=False),
        scratch_types=[
            pltpu.VMEM((max(SEQ_SC0, SEQ_SC1), SEQ), i32),
            pltpu.VMEM((SEQ, H // 2), i32),
            pltpu.VMEM((SEQ, H // 2), i32),
            pltpu.VMEM((SEQ, H // 2), i32),
            pltpu.VMEM((SEQ, H // 2), i32),
            pltpu.VMEM((OUT_SUB, H), f32),
            pltpu.VMEM((OUT_SUB, H), f32),
            pltpu.VMEM((DEG_K, BATCH), i32),
            pltpu.VMEM((BATCH,), f32),
            pltpu.VMEM((DEG_SLICE,), f32),
            pltpu.SemaphoreType.DMA,
            pltpu.SemaphoreType.DMA,
            pltpu.SemaphoreType.DMA,
            pltpu.SemaphoreType.DMA,
            pltpu.SemaphoreType.DMA,
            pltpu.SemaphoreType.DMA,
            pltpu.SemaphoreType.DMA,
            pltpu.VMEM_SHARED((DEG_TOT,), f32),
        ],
    )


def _sc_msg_body(h_hbm, sidx_hbm, didx_hbm,
                 accp_hbm,
                 si_v, di_v, msg0, msg1, accbuf,
                 gsem0, gsem1, ssem0, ssem1, acc_sh):
    cid = lax.axis_index("c")
    sid = lax.axis_index("s")
    wid = sid * NC + cid

    zero16 = jnp.zeros((L,), f32)

    def zfill(r, carry):
        accbuf[r, :] = zero16
        return carry

    lax.fori_loop(0, ACC_SLICE, zfill, 0)
    pltpu.sync_copy(accbuf, acc_sh.at[pl.ds(sid * ACC_SLICE, ACC_SLICE)])
    plsc.subcore_barrier()

    pltpu.sync_copy(sidx_hbm.at[wid], si_v)
    pltpu.sync_copy(didx_hbm.at[wid], di_v)

    msgs = [msg0, msg1]
    gsems = [gsem0, gsem1]
    ssems = [ssem0, ssem1]
    pltpu.async_copy(h_hbm.at[si_v.at[0]], msg0, gsem0)
    pltpu.async_copy(h_hbm.at[si_v.at[1]], msg1, gsem1)

    def body(g, carry):
        for b in range(2):
            j = g * 2 + b
            msg = msgs[b]
            pltpu.make_async_copy(h_hbm.at[si_v.at[j]], msg, gsems[b]).wait()
            pltpu.async_copy(msg, acc_sh.at[di_v.at[j]], ssems[b], add=True)

            @pl.when(j + 2 < MSG_K)
            def _():
                pltpu.make_async_copy(msg, acc_sh.at[di_v.at[j]],
                                      ssems[b]).wait()
                pltpu.async_copy(h_hbm.at[si_v.at[j + 2]], msg, gsems[b])
        return carry

    lax.fori_loop(0, MSG_K // 2, body, 0)
    pltpu.make_async_copy(msg0, acc_sh.at[di_v.at[MSG_K - 2]], ssem0).wait()
    pltpu.make_async_copy(msg1, acc_sh.at[di_v.at[MSG_K - 1]], ssem1).wait()
    plsc.subcore_barrier()
    pltpu.sync_copy(acc_sh.at[pl.ds(sid * ACC_SLICE, ACC_SLICE)], accbuf)
    pltpu.sync_copy(accbuf,
                    accp_hbm.at[pl.ds(cid * ACC_ROWS + sid * ACC_SLICE,
                                      ACC_SLICE)])


def _make_sc_msg():
    mesh = plsc.VectorSubcoreMesh(core_axis_name="c", subcore_axis_name="s")
    return pl.kernel(
        _sc_msg_body,
        out_type=jax.ShapeDtypeStruct((NC * ACC_ROWS, C), f32),
        mesh=mesh,
        compiler_params=pltpu.CompilerParams(use_tc_tiling_on_sc=False),
        scratch_types=[
            pltpu.VMEM((MSG_K, BATCH), i32),
            pltpu.VMEM((MSG_K, BATCH), i32),
            pltpu.VMEM((BATCH, C), f32),
            pltpu.VMEM((BATCH, C), f32),
            pltpu.VMEM((ACC_SLICE, C), f32),
            pltpu.SemaphoreType.DMA,
            pltpu.SemaphoreType.DMA,
            pltpu.SemaphoreType.DMA,
            pltpu.SemaphoreType.DMA,
            pltpu.VMEM_SHARED((ACC_ROWS, C), f32),
        ],
    )


CAST_BLK = 2048


def _tc_cast_body(x_ref, out_ref):
    xb = x_ref[...].astype(jnp.bfloat16)
    lo16 = lax.bitcast_convert_type(xb[:, 0:H // 2], jnp.int16)
    hi16 = lax.bitcast_convert_type(xb[:, H // 2:H], jnp.int16)
    lo = lo16.astype(i32) & 0xFFFF
    hi = lax.shift_left(hi16.astype(i32), 16)
    out_ref[...] = lo | hi


def _make_tc_cast(n):
    grid = (n + CAST_BLK - 1) // CAST_BLK
    return pl.pallas_call(
        _tc_cast_body,
        grid=(grid,),
        in_specs=[pl.BlockSpec((CAST_BLK, H), lambda i: (i, 0))],
        out_specs=pl.BlockSpec((CAST_BLK, H // 2), lambda i: (i, 0)),
        out_shape=jax.ShapeDtypeStruct((n, H // 2), i32),
    )


B1_BLK = 448


def _rsqrt_deg(dp_ref):
    return lax.rsqrt(jnp.maximum(dp_ref[0] + dp_ref[1], 1.0))


def _tc_pool_body(x_ref, wp_ref, bp_ref, wcc_ref, wtc_ref, dcc_ref, dtc_ref,
                  hcc_ref, htc_ref):
    t = jnp.dot(x_ref[...], wp_ref[...], preferred_element_type=f32)
    t = jnp.tanh(t * (1.0 / SEQ) + bp_ref[...])
    hcc_ref[...] = jnp.dot(t, wcc_ref[...],
                           preferred_element_type=f32) * _rsqrt_deg(dcc_ref)
    htc_ref[...] = jnp.dot(t, wtc_ref[...],
                           preferred_element_type=f32) * _rsqrt_deg(dtc_ref)


def _make_tc_pool():
    d_spec = pl.BlockSpec((2, B1_BLK, 1), lambda i: (0, i, 0))
    return pl.pallas_call(
        _tc_pool_body,
        grid=(N_SEQ_PAD // B1_BLK,),
        in_specs=[pl.BlockSpec((B1_BLK, H), lambda i: (i, 0)),
                  pl.BlockSpec((H, H), lambda i: (0, 0)),
                  pl.BlockSpec((1, H), lambda i: (0, 0)),
                  pl.BlockSpec((H, C), lambda i: (0, 0)),
                  pl.BlockSpec((H, C), lambda i: (0, 0)),
                  d_spec, d_spec],
        out_specs=[pl.BlockSpec((B1_BLK, C), lambda i: (i, 0)),
                   pl.BlockSpec((B1_BLK, C), lambda i: (i, 0))],
        out_shape=[jax.ShapeDtypeStruct((N_SEQ_PAD, C), f32),
                   jax.ShapeDtypeStruct((N_SEQ_PAD, C), f32)],
    )


B2_BLK = 1000


def _tc_num_body(x_ref, wf_ref, wnc_ref, bf_ref, dnum_ref, out_ref):
    w2 = jnp.dot(wf_ref[...], wnc_ref[...], preferred_element_type=f32)
    c2 = jnp.dot(bf_ref[...], wnc_ref[...], preferred_element_type=f32)
    g = jnp.dot(x_ref[...], w2, preferred_element_type=f32) + c2
    out_ref[...] = g * _rsqrt_deg(dnum_ref)


def _make_tc_num():
    return pl.pallas_call(
        _tc_num_body,
        grid=(N_NUM // B2_BLK,),
        in_specs=[pl.BlockSpec((B2_BLK, 192), lambda i: (i, 0)),
                  pl.BlockSpec((192, H), lambda i: (0, 0)),
                  pl.BlockSpec((H, C), lambda i: (0, 0)),
                  pl.BlockSpec((1, H), lambda i: (0, 0)),
                  pl.BlockSpec((2, B2_BLK, 1), lambda i: (0, i, 0))],
        out_specs=pl.BlockSpec((B2_BLK, C), lambda i: (i, 0)),
        out_shape=jax.ShapeDtypeStruct((N_NUM, C), f32),
    )


D_BLK = 2000


def _tc_final_body(mtc_ref, mcc_ref, mnc_ref, dtc_ref, dcc_ref, dnc_ref,
                   btc_ref, bcc_ref, bnc_ref, out_ref):
    out_ref[...] = ((mtc_ref[0] + mtc_ref[1]) * _rsqrt_deg(dtc_ref)
                    + (mcc_ref[0] + mcc_ref[1]) * _rsqrt_deg(dcc_ref)
                    + (mnc_ref[0] + mnc_ref[1]) * _rsqrt_deg(dnc_ref)
                    + btc_ref[...] + bcc_ref[...] + bnc_ref[...])


def _make_tc_final():
    m_spec = pl.BlockSpec((2, D_BLK, C), lambda i: (0, i, 0))
    d_spec = pl.BlockSpec((2, D_BLK, 1), lambda i: (0, i, 0))
    b_spec = pl.BlockSpec((1, C), lambda i: (0, 0))
    return pl.pallas_call(
        _tc_final_body,
        grid=(N_COL // D_BLK,),
        in_specs=[m_spec, m_spec, m_spec, d_spec, d_spec, d_spec,
                  b_spec, b_spec, b_spec],
        out_specs=pl.BlockSpec((D_BLK, C), lambda i: (i, 0)),
        out_shape=jax.ShapeDtypeStruct((N_COL, C), f32),
    )


def kernel(column_input_ids, table_input_ids, num_feat,
           edge_index_tc, edge_index_cc, edge_index_nc,
           emb_table, W_pool, b_pool, W_feat, b_feat,
           W_tc, b_tc, W_cc, b_cc, W_nc, b_nc):
    ids_all = jnp.concatenate([column_input_ids, table_input_ids], axis=0)
    ids_all = jnp.concatenate(
        [ids_all, jnp.zeros((N_SEQ_PAD - N_SEQ, SEQ), ids_all.dtype)], axis=0)
    ids2 = ids_all.astype(i32)

    tc_s, tc_d = edge_index_tc[0], edge_index_tc[1]
    cc_s, cc_d = edge_index_cc[0], edge_index_cc[1]
    nc_s, nc_d = edge_index_nc[0], edge_index_nc[1]

    deg_idx = jnp.concatenate([
        tc_s + OFF_TAB, cc_s + OFF_COL, nc_s + OFF_NUM,
        tc_d + OFF_ITC, cc_d + OFF_ICC, nc_d + OFF_INC,
        jnp.full((NW * DEG_K * BATCH - E_DEG,), DEG_TRASH, i32),
    ]).astype(i32).reshape(NW, DEG_K, BATCH)

    msg_pad = NW * MSG_K * BATCH - E_MSG
    msg_src = jnp.concatenate([
        tc_s + OFF_TAB, cc_s + OFF_COL, nc_s + OFF_NUM,
        jnp.zeros((msg_pad,), i32),
    ]).astype(i32).reshape(NW, MSG_K, BATCH)
    msg_dst = jnp.concatenate([
        tc_d, cc_d + ACC_RELOFF, nc_d + 2 * ACC_RELOFF,
        jnp.full((msg_pad,), ACC_TRASH, i32),
    ]).astype(i32).reshape(NW, MSG_K, BATCH)

    emb_i32 = _make_tc_cast(emb_table.shape[0])(emb_table)

    pooled_sum, degp_flat = _make_sc_pool_deg()(ids2, emb_i32, deg_idx)
    degp = degp_flat.reshape(NC, DEG_TOT)

    dcc_vec = degp[:, OFF_COL:OFF_COL + N_SEQ_PAD, None]
    dtc_vec = jnp.concatenate(
        [jnp.zeros((NC, N_COL), f32),
         degp[:, OFF_TAB:OFF_TAB + N_SEQ_PAD - N_COL]], axis=1)[..., None]
    hcc, htc = _make_tc_pool()(pooled_sum, W_pool, b_pool.reshape(1, H),
                               W_cc, W_tc, dcc_vec, dtc_vec)
    dnum_vec = degp[:, OFF_NUM:OFF_NUM + N_NUM, None]
    h_num = _make_tc_num()(num_feat, W_feat, W_nc, b_feat.reshape(1, H),
                           dnum_vec)

    h_all = jnp.concatenate(
        [htc[N_COL:N_COL + 512], hcc[:N_COL], h_num], axis=0)

    accp = _make_sc_msg()(h_all, msg_src, msg_dst).reshape(
        NC, ACC_ROWS, C)

    dtc = degp[:, OFF_ITC:OFF_ITC + N_COL, None]
    dcc = degp[:, OFF_ICC:OFF_ICC + N_COL, None]
    dnc = degp[:, OFF_INC:OFF_INC + N_COL, None]
    m_tc = accp[:, 0:N_COL]
    m_cc = accp[:, ACC_RELOFF:ACC_RELOFF + N_COL]
    m_nc = accp[:, 2 * ACC_RELOFF:2 * ACC_RELOFF + N_COL]
    out = _make_tc_final()(m_tc, m_cc, m_nc, dtc, dcc, dnc,
                           b_tc.reshape(1, C), b_cc.reshape(1, C),
                           b_nc.reshape(1, C))
    return out


# speedup vs baseline: 1.0418x; 1.0418x over previous
import jax
import jax.numpy as jnp
from jax import lax
from jax.experimental import pallas as pl
from jax.experimental.pallas import tpu as pltpu
from jax.experimental.pallas import tpu_sc as plsc

N_COL = 10000
N_TAB = 500
N_NUM = 30000
SEQ = 16
H = 768
C = 16
E_TC = 10000
E_CC = 64000
E_NC = 30000

NC, NS, L = 2, 16, 16
NW = NC * NS

N_SEQ = N_COL + N_TAB
N_SEQ_PAD = 10752
OUT_SUB = 16
NBUF = 4
SEQ_SC0 = 384
SEQ_SC1 = 288

OFF_TAB = 0
OFF_COL = 512
OFF_NUM = 10512
OFF_ITC = 40512
OFF_ICC = 50512
OFF_INC = 60512
DEG_TOT = 70656
DEG_SLICE = DEG_TOT // NS
DEG_TRASH = 70600

BATCH = 128
E_DEG = 2 * (E_TC + E_CC + E_NC)
DEG_K = 51

H_ROWS = 40512
ACC_RELOFF = 10112
ACC_ROWS = 3 * ACC_RELOFF
ACC_TRASH = 10008
ACC_SLICE = ACC_ROWS // NS
E_MSG = E_TC + E_CC + E_NC
MSG_K = 26

f32 = jnp.float32
i32 = jnp.int32


def _sc_deg_body(degidx_hbm, degp_hbm, didx_v, ones_v, degbuf, dsem, deg_sh):
    cid = lax.axis_index("c")
    sid = lax.axis_index("s")
    wid = sid * NC + cid

    zero16 = jnp.zeros((L,), f32)

    def zfill(r, carry):
        degbuf[pl.ds(r * L, L)] = zero16
        return carry

    lax.fori_loop(0, DEG_SLICE // L, zfill, 0)
    pltpu.sync_copy(degbuf, deg_sh.at[pl.ds(sid * DEG_SLICE, DEG_SLICE)])
    plsc.subcore_barrier()

    ones16 = jnp.full((L,), 1.0, f32)
    for i in range(BATCH // L):
        ones_v[pl.ds(i * L, L)] = ones16

    pltpu.sync_copy(degidx_hbm.at[wid], didx_v)
    for j in range(DEG_K):
        pltpu.async_copy(ones_v, deg_sh.at[didx_v.at[j]], dsem, add=True)
    for j in range(DEG_K):
        pltpu.make_async_copy(ones_v, deg_sh.at[didx_v.at[j]], dsem).wait()
    plsc.subcore_barrier()
    pltpu.sync_copy(deg_sh.at[pl.ds(sid * DEG_SLICE, DEG_SLICE)], degbuf)
    pltpu.sync_copy(degbuf,
                    degp_hbm.at[pl.ds(cid * DEG_TOT + sid * DEG_SLICE,
                                      DEG_SLICE)])


def _make_sc_deg():
    mesh = plsc.VectorSubcoreMesh(core_axis_name="c", subcore_axis_name="s")
    return pl.kernel(
        _sc_deg_body,
        out_type=jax.ShapeDtypeStruct((NC * DEG_TOT,), f32),
        mesh=mesh,
        scratch_types=[
            pltpu.VMEM((DEG_K, BATCH), i32),
            pltpu.VMEM((BATCH,), f32),
            pltpu.VMEM((DEG_SLICE,), f32),
            pltpu.SemaphoreType.DMA,
            pltpu.VMEM_SHARED((DEG_TOT,), f32),
        ],
    )


def _sc_pool_body(ids_hbm, emb_hbm,
                  out_hbm,
                  idx_v, tok0, tok1, tok2, tok3, obuf0, obuf1,
                  sem0, sem1, sem2, sem3, osem0, osem1):
    cid = lax.axis_index("c")
    sid = lax.axis_index("s")

    toks = [tok0, tok1, tok2, tok3]
    sems = [sem0, sem1, sem2, sem3]
    obufs = [obuf0, obuf1]
    osems = [osem0, osem1]
    himask = jnp.full((L,), -65536, i32)
    sh16 = jnp.full((L,), 16, i32)

    def pool_part(base, nseq):
        n_pairs = nseq // (2 * OUT_SUB)
        pltpu.sync_copy(ids_hbm.at[pl.ds(base, nseq)],
                        idx_v.at[pl.ds(0, nseq)])
        for b in range(NBUF):
            pltpu.async_copy(emb_hbm.at[idx_v.at[b]], toks[b], sems[b])

        def pair_body(p, carry):
            for e in range(2):
                outbuf = obufs[e]
                osem = osems[e]

                @pl.when(p >= 1)
                def _():
                    pltpu.make_async_copy(
                        outbuf, out_hbm.at[pl.ds(base, OUT_SUB)],
                        osem).wait()

                def quad_body(g, carry2, e=e, outbuf=outbuf):
                    for b in range(NBUF):
                        s = (p * 2 + e) * OUT_SUB + g * NBUF + b
                        tok = toks[b]
                        sem = sems[b]
                        pltpu.make_async_copy(emb_hbm.at[idx_v.at[s]], tok,
                                              sem).wait()
                        srow = g * NBUF + b

                        def col_body(k, carry3):
                            acc_lo = jnp.zeros((L,), f32)
                            acc_hi = jnp.zeros((L,), f32)
                            for j in range(SEQ):
                                vi = tok[j, pl.ds(k * L, L)]
                                acc_lo = acc_lo + plsc.bitcast(
                                    lax.shift_left(vi, sh16), f32)
                                acc_hi = acc_hi + plsc.bitcast(
                                    vi & himask, f32)
                            outbuf[srow, pl.ds(k * L, L)] = acc_lo
                            outbuf[srow, pl.ds(H // 2 + k * L, L)] = acc_hi
                            return carry3

                        lax.fori_loop(0, H // (2 * L), col_body, 0)

                        @pl.when(s + NBUF < nseq)
                        def _():
                            pltpu.async_copy(
                                emb_hbm.at[idx_v.at[s + NBUF]], tok, sem)
                    return carry2

                lax.fori_loop(0, OUT_SUB // NBUF, quad_body, 0)
                pltpu.async_copy(
                    outbuf,
                    out_hbm.at[pl.ds(base + (p * 2 + e) * OUT_SUB,
                                     OUT_SUB)],
                    osem)
            return carry

        lax.fori_loop(0, n_pairs, pair_body, 0)
        for e in range(2):
            pltpu.make_async_copy(obufs[e],
                                  out_hbm.at[pl.ds(base, OUT_SUB)],
                                  osems[e]).wait()

    @pl.when(cid == 0)
    def _():
        pool_part(sid * SEQ_SC0, SEQ_SC0)

    @pl.when(cid == 1)
    def _():
        pool_part(NS * SEQ_SC0 + sid * SEQ_SC1, SEQ_SC1)


def _make_sc_pool():
    mesh = plsc.VectorSubcoreMesh(core_axis_name="c", subcore_axis_name="s")
    return pl.kernel(
        _sc_pool_body,
        out_type=jax.ShapeDtypeStruct((N_SEQ_PAD, H), f32),
        mesh=mesh,
        compiler_params=pltpu.CompilerParams(needs_layout_passes=False),
        scratch_types=[
            pltpu.VMEM((max(SEQ_SC0, SEQ_SC1), SEQ), i32),
            pltpu.VMEM((SEQ, H // 2), i32),
            pltpu.VMEM((SEQ, H // 2), i32),
            pltpu.VMEM((SEQ, H // 2), i32),
            pltpu.VMEM((SEQ, H // 2), i32),
            pltpu.VMEM((OUT_SUB, H), f32),
            pltpu.VMEM((OUT_SUB, H), f32),
            pltpu.SemaphoreType.DMA,
            pltpu.SemaphoreType.DMA,
            pltpu.SemaphoreType.DMA,
            pltpu.SemaphoreType.DMA,
            pltpu.SemaphoreType.DMA,
            pltpu.SemaphoreType.DMA,
        ],
    )


def _sc_msg_body(h_hbm, sidx_hbm, didx_hbm,
                 accp_hbm,
                 si_v, di_v, msg0, msg1, accbuf,
                 gsem0, gsem1, ssem0, ssem1, acc_sh):
    cid = lax.axis_index("c")
    sid = lax.axis_index("s")
    wid = sid * NC + cid

    zero16 = jnp.zeros((L,), f32)

    def zfill(r, carry):
        accbuf[r, :] = zero16
        return carry

    lax.fori_loop(0, ACC_SLICE, zfill, 0)
    pltpu.sync_copy(accbuf, acc_sh.at[pl.ds(sid * ACC_SLICE, ACC_SLICE)])
    plsc.subcore_barrier()

    pltpu.sync_copy(sidx_hbm.at[wid], si_v)
    pltpu.sync_copy(didx_hbm.at[wid], di_v)

    msgs = [msg0, msg1]
    gsems = [gsem0, gsem1]
    ssems = [ssem0, ssem1]
    pltpu.async_copy(h_hbm.at[si_v.at[0]], msg0, gsem0)
    pltpu.async_copy(h_hbm.at[si_v.at[1]], msg1, gsem1)

    def body(g, carry):
        for b in range(2):
            j = g * 2 + b
            msg = msgs[b]
            pltpu.make_async_copy(h_hbm.at[si_v.at[j]], msg, gsems[b]).wait()
            pltpu.async_copy(msg, acc_sh.at[di_v.at[j]], ssems[b], add=True)

            @pl.when(j + 2 < MSG_K)
            def _():
                pltpu.make_async_copy(msg, acc_sh.at[di_v.at[j]],
                                      ssems[b]).wait()
                pltpu.async_copy(h_hbm.at[si_v.at[j + 2]], msg, gsems[b])
        return carry

    lax.fori_loop(0, MSG_K // 2, body, 0)
    pltpu.make_async_copy(msg0, acc_sh.at[di_v.at[MSG_K - 2]], ssem0).wait()
    pltpu.make_async_copy(msg1, acc_sh.at[di_v.at[MSG_K - 1]], ssem1).wait()
    plsc.subcore_barrier()
    pltpu.sync_copy(acc_sh.at[pl.ds(sid * ACC_SLICE, ACC_SLICE)], accbuf)
    pltpu.sync_copy(accbuf,
                    accp_hbm.at[pl.ds(cid * ACC_ROWS + sid * ACC_SLICE,
                                      ACC_SLICE)])


def _make_sc_msg():
    mesh = plsc.VectorSubcoreMesh(core_axis_name="c", subcore_axis_name="s")
    return pl.kernel(
        _sc_msg_body,
        out_type=jax.ShapeDtypeStruct((NC * ACC_ROWS, C), f32),
        mesh=mesh,
        compiler_params=pltpu.CompilerParams(use_tc_tiling_on_sc=False),
        scratch_types=[
            pltpu.VMEM((MSG_K, BATCH), i32),
            pltpu.VMEM((MSG_K, BATCH), i32),
            pltpu.VMEM((BATCH, C), f32),
            pltpu.VMEM((BATCH, C), f32),
            pltpu.VMEM((ACC_SLICE, C), f32),
            pltpu.SemaphoreType.DMA,
            pltpu.SemaphoreType.DMA,
            pltpu.SemaphoreType.DMA,
            pltpu.SemaphoreType.DMA,
            pltpu.VMEM_SHARED((ACC_ROWS, C), f32),
        ],
    )


CAST_BLK = 1024
B2_BLK = 1000
B1_BLK = 448


def _rsqrt_deg(dp_ref):
    return lax.rsqrt(jnp.maximum(dp_ref[0] + dp_ref[1], 1.0))


def _tc_pool_body(x_ref, wp_ref, bp_ref, wcc_ref, wtc_ref, dcc_ref, dtc_ref,
                  hcc_ref, htc_ref):
    t = jnp.dot(x_ref[...], wp_ref[...], preferred_element_type=f32)
    t = jnp.tanh(t * (1.0 / SEQ) + bp_ref[...])
    hcc_ref[...] = jnp.dot(t, wcc_ref[...],
                           preferred_element_type=f32) * _rsqrt_deg(dcc_ref)
    htc_ref[...] = jnp.dot(t, wtc_ref[...],
                           preferred_element_type=f32) * _rsqrt_deg(dtc_ref)


def _make_tc_pool():
    d_spec = pl.BlockSpec((2, B1_BLK, 1), lambda i: (0, i, 0))
    return pl.pallas_call(
        _tc_pool_body,
        grid=(N_SEQ_PAD // B1_BLK,),
        in_specs=[pl.BlockSpec((B1_BLK, H), lambda i: (i, 0)),
                  pl.BlockSpec((H, H), lambda i: (0, 0)),
                  pl.BlockSpec((1, H), lambda i: (0, 0)),
                  pl.BlockSpec((H, C), lambda i: (0, 0)),
                  pl.BlockSpec((H, C), lambda i: (0, 0)),
                  d_spec, d_spec],
        out_specs=[pl.BlockSpec((B1_BLK, C), lambda i: (i, 0)),
                   pl.BlockSpec((B1_BLK, C), lambda i: (i, 0))],
        out_shape=[jax.ShapeDtypeStruct((N_SEQ_PAD, C), f32),
                   jax.ShapeDtypeStruct((N_SEQ_PAD, C), f32)],
    )


def _tc_cast_body(x_ref, out_ref):
    xb = x_ref[...].astype(jnp.bfloat16)
    lo16 = lax.bitcast_convert_type(xb[:, 0:H // 2], jnp.int16)
    hi16 = lax.bitcast_convert_type(xb[:, H // 2:H], jnp.int16)
    lo = lo16.astype(i32) & 0xFFFF
    hi = lax.shift_left(hi16.astype(i32), 16)
    out_ref[...] = lo | hi


def _make_tc_cast(n):
    grid = (n + CAST_BLK - 1) // CAST_BLK
    return pl.pallas_call(
        _tc_cast_body,
        grid=(grid,),
        in_specs=[pl.BlockSpec((CAST_BLK, H), lambda i: (i, 0))],
        out_specs=pl.BlockSpec((CAST_BLK, H // 2), lambda i: (i, 0)),
        out_shape=jax.ShapeDtypeStruct((n, H // 2), i32),
    )


def _tc_num_body(x_ref, wf_ref, wnc_ref, bf_ref, dnum_ref, out_ref):
    w2 = jnp.dot(wf_ref[...], wnc_ref[...], preferred_element_type=f32)
    c2 = jnp.dot(bf_ref[...], wnc_ref[...], preferred_element_type=f32)
    g = jnp.dot(x_ref[...], w2, preferred_element_type=f32) + c2
    out_ref[...] = g * _rsqrt_deg(dnum_ref)


def _make_tc_num():
    return pl.pallas_call(
        _tc_num_body,
        grid=(N_NUM // B2_BLK,),
        in_specs=[pl.BlockSpec((B2_BLK, 192), lambda i: (i, 0)),
                  pl.BlockSpec((192, H), lambda i: (0, 0)),
                  pl.BlockSpec((H, C), lambda i: (0, 0)),
                  pl.BlockSpec((1, H), lambda i: (0, 0)),
                  pl.BlockSpec((2, B2_BLK, 1), lambda i: (0, i, 0))],
        out_specs=pl.BlockSpec((B2_BLK, C), lambda i: (i, 0)),
        out_shape=jax.ShapeDtypeStruct((N_NUM, C), f32),
    )


D_BLK = 2000


def _tc_final_body(mtc_ref, mcc_ref, mnc_ref, dtc_ref, dcc_ref, dnc_ref,
                   btc_ref, bcc_ref, bnc_ref, out_ref):
    out_ref[...] = ((mtc_ref[0] + mtc_ref[1]) * _rsqrt_deg(dtc_ref)
                    + (mcc_ref[0] + mcc_ref[1]) * _rsqrt_deg(dcc_ref)
                    + (mnc_ref[0] + mnc_ref[1]) * _rsqrt_deg(dnc_ref)
                    + btc_ref[...] + bcc_ref[...] + bnc_ref[...])


def _make_tc_final():
    m_spec = pl.BlockSpec((2, D_BLK, C), lambda i: (0, i, 0))
    d_spec = pl.BlockSpec((2, D_BLK, 1), lambda i: (0, i, 0))
    b_spec = pl.BlockSpec((1, C), lambda i: (0, 0))
    return pl.pallas_call(
        _tc_final_body,
        grid=(N_COL // D_BLK,),
        in_specs=[m_spec, m_spec, m_spec, d_spec, d_spec, d_spec,
                  b_spec, b_spec, b_spec],
        out_specs=pl.BlockSpec((D_BLK, C), lambda i: (i, 0)),
        out_shape=jax.ShapeDtypeStruct((N_COL, C), f32),
    )


def kernel(column_input_ids, table_input_ids, num_feat,
           edge_index_tc, edge_index_cc, edge_index_nc,
           emb_table, W_pool, b_pool, W_feat, b_feat,
           W_tc, b_tc, W_cc, b_cc, W_nc, b_nc):
    ids_all = jnp.concatenate([column_input_ids, table_input_ids], axis=0)
    ids_all = jnp.concatenate(
        [ids_all, jnp.zeros((N_SEQ_PAD - N_SEQ, SEQ), ids_all.dtype)], axis=0)
    ids2 = ids_all.astype(i32)

    tc_s, tc_d = edge_index_tc[0], edge_index_tc[1]
    cc_s, cc_d = edge_index_cc[0], edge_index_cc[1]
    nc_s, nc_d = edge_index_nc[0], edge_index_nc[1]

    deg_idx = jnp.concatenate([
        tc_s + OFF_TAB, cc_s + OFF_COL, nc_s + OFF_NUM,
        tc_d + OFF_ITC, cc_d + OFF_ICC, nc_d + OFF_INC,
        jnp.full((NW * DEG_K * BATCH - E_DEG,), DEG_TRASH, i32),
    ]).astype(i32).reshape(NW, DEG_K, BATCH)

    msg_pad = NW * MSG_K * BATCH - E_MSG
    msg_src = jnp.concatenate([
        tc_s + OFF_TAB, cc_s + OFF_COL, nc_s + OFF_NUM,
        jnp.zeros((msg_pad,), i32),
    ]).astype(i32).reshape(NW, MSG_K, BATCH)
    msg_dst = jnp.concatenate([
        tc_d, cc_d + ACC_RELOFF, nc_d + 2 * ACC_RELOFF,
        jnp.full((msg_pad,), ACC_TRASH, i32),
    ]).astype(i32).reshape(NW, MSG_K, BATCH)

    emb_i32 = _make_tc_cast(emb_table.shape[0])(emb_table)

    degp = _make_sc_deg()(deg_idx).reshape(NC, DEG_TOT)

    pooled_sum = _make_sc_pool()(ids2, emb_i32)

    dcc_vec = degp[:, OFF_COL:OFF_COL + N_SEQ_PAD, None]
    dtc_vec = jnp.concatenate(
        [jnp.zeros((NC, N_COL), f32),
         degp[:, OFF_TAB:OFF_TAB + N_SEQ_PAD - N_COL]], axis=1)[..., None]
    hcc, htc = _make_tc_pool()(pooled_sum, W_pool, b_pool.reshape(1, H),
                               W_cc, W_tc, dcc_vec, dtc_vec)
    dnum_vec = degp[:, OFF_NUM:OFF_NUM + N_NUM, None]
    h_num = _make_tc_num()(num_feat, W_feat, W_nc, b_feat.reshape(1, H),
                           dnum_vec)

    h_all = jnp.concatenate(
        [htc[N_COL:N_COL + 512], hcc[:N_COL], h_num], axis=0)

    accp = _make_sc_msg()(h_all, msg_src, msg_dst).reshape(
        NC, ACC_ROWS, C)

    dtc = degp[:, OFF_ITC:OFF_ITC + N_COL, None]
    dcc = degp[:, OFF_ICC:OFF_ICC + N_COL, None]
    dnc = degp[:, OFF_INC:OFF_INC + N_COL, None]
    m_tc = accp[:, 0:N_COL]
    m_cc = accp[:, ACC_RELOFF:ACC_RELOFF + N_COL]
    m_nc = accp[:, 2 * ACC_RELOFF:2 * ACC_RELOFF + N_COL]
    out = _make_tc_final()(m_tc, m_cc, m_nc, dtc, dcc, dnc,
                           b_tc.reshape(1, C), b_cc.reshape(1, C),
                           b_nc.reshape(1, C))
    return out
